# baseline (speedup 1.0000x reference)
import functools

import jax
import jax.numpy as jnp
from jax import lax
from jax.experimental import pallas as pl
from jax.experimental.pallas import tpu as pltpu
from jax.experimental.pallas import tpu_sc as plsc

_DIM = 64
_IDX_LANES = 128
_BBLK = 128


@functools.cache
def _build_gather(n_total):
    info = plsc.get_sparse_core_info()
    nc, ns = info.num_cores, info.num_subcores
    nw = nc * ns
    rows_per_w = n_total // nw
    idx_rows_per_w = rows_per_w // _IDX_LANES
    n_grp = rows_per_w // 16

    mesh = plsc.VectorSubcoreMesh(core_axis_name="c", subcore_axis_name="s")

    @functools.partial(
        pl.kernel,
        mesh=mesh,
        compiler_params=pltpu.CompilerParams(
            use_tc_tiling_on_sc=True, needs_layout_passes=False
        ),
        out_type=jax.ShapeDtypeStruct((n_total, _DIM), jnp.float32),
        scratch_types=[
            pltpu.VMEM((idx_rows_per_w, _IDX_LANES), jnp.int32),
            pltpu.SemaphoreType.DMA,
        ],
    )
    def gather_kernel(idx_hbm, table_hbm, out_hbm, idx_all, gsem):
        wid = lax.axis_index("s") * nc + lax.axis_index("c")
        idx_row0 = wid * idx_rows_per_w
        out_row0 = wid * rows_per_w
        pltpu.sync_copy(idx_hbm.at[pl.ds(idx_row0, idx_rows_per_w)], idx_all)

        @pl.loop(0, n_grp)
        def _grp(g):
            vec = idx_all[g // 8, pl.ds((g % 8) * 16, 16)]
            row0 = out_row0 + g * 16
            for i in range(16):
                pltpu.async_copy(
                    table_hbm.at[pl.ds(vec[i], 1)],
                    out_hbm.at[pl.ds(row0 + i, 1)],
                    gsem,
                )

        pltpu.make_async_copy(
            out_hbm.at[pl.ds(out_row0, rows_per_w)],
            out_hbm.at[pl.ds(out_row0, rows_per_w)],
            gsem,
        ).wait()

    return gather_kernel


@functools.cache
def _build_transpose(batch, fields):
    def body(rows_ref, out_ref):
        t = jnp.transpose(rows_ref[...])
        out_ref[...] = t.reshape(1, _DIM // 8, 1, 8, _BBLK)

    return pl.pallas_call(
        body,
        grid=(fields, batch // _BBLK),
        in_specs=[
            pl.BlockSpec(
                (_BBLK, _DIM), lambda f, i: (f * (batch // _BBLK) + i, 0)
            ),
        ],
        out_specs=pl.BlockSpec(
            (1, _DIM // 8, 1, 8, _BBLK), lambda f, i: (f, 0, i, 0, 0)
        ),
        out_shape=jax.ShapeDtypeStruct(
            (fields, _DIM // 8, batch // _BBLK, 8, _BBLK), jnp.float32
        ),
    )


def kernel(x, weight):
    b, f = x.shape
    n_total = b * f
    xt = jnp.swapaxes(x, 0, 1).astype(jnp.int32)
    idx2d = xt.reshape(n_total // _IDX_LANES, _IDX_LANES)
    rows = _build_gather(n_total)(idx2d, weight)
    out5 = _build_transpose(b, f)(rows)
    return jnp.transpose(out5, (2, 4, 0, 1, 3)).reshape(b, f, _DIM)


# speedup vs baseline: 3.8417x; 3.8417x over previous
import functools

import jax
import jax.numpy as jnp
from jax import lax
from jax.experimental import pallas as pl
from jax.experimental.pallas import tpu as pltpu
from jax.experimental.pallas import tpu_sc as plsc

_DIM = 64
_IDX_LANES = 128
_CHUNK = 128
_NBUF = 4
_BBLK = 128


@functools.cache
def _build_gather(n_total):
    info = plsc.get_sparse_core_info()
    nc, ns = info.num_cores, info.num_subcores
    nw = nc * ns
    rows_per_w = n_total // nw
    n_chunks = rows_per_w // _CHUNK
    idx_rows_per_w = rows_per_w // _IDX_LANES

    mesh = plsc.VectorSubcoreMesh(core_axis_name="c", subcore_axis_name="s")

    @functools.partial(
        pl.kernel,
        mesh=mesh,
        compiler_params=pltpu.CompilerParams(use_tc_tiling_on_sc=True),
        out_type=jax.ShapeDtypeStruct((n_total, _DIM), jnp.float32),
        scratch_types=[
            pltpu.VMEM((idx_rows_per_w, _IDX_LANES), jnp.int32),
            pltpu.VMEM((_NBUF, _CHUNK, _DIM), jnp.float32),
            pltpu.SemaphoreType.DMA((_NBUF,)),
            pltpu.SemaphoreType.DMA((_NBUF,)),
        ],
    )
    def gather_kernel(idx_hbm, table_hbm, out_hbm, idx_all, rows_v, gsem, ssem):
        wid = lax.axis_index("s") * nc + lax.axis_index("c")
        idx_row0 = wid * idx_rows_per_w
        out_row0 = wid * rows_per_w
        pltpu.sync_copy(idx_hbm.at[pl.ds(idx_row0, idx_rows_per_w)], idx_all)

        def start_gathers(c, b):
            @pl.loop(0, _CHUNK // 16)
            def _grp(g):
                vec = idx_all[c, pl.ds(g * 16, 16)]
                for i in range(16):
                    pltpu.async_copy(
                        table_hbm.at[pl.ds(vec[i], 1)],
                        rows_v.at[b, pl.ds(g * 16 + i, 1)],
                        gsem.at[b],
                    )

        def wait_gathers(b):
            pltpu.make_async_copy(
                table_hbm.at[pl.ds(0, _CHUNK)], rows_v.at[b], gsem.at[b]
            ).wait()

        def start_store(c, b):
            return pltpu.async_copy(
                rows_v.at[b],
                out_hbm.at[pl.ds(out_row0 + c * _CHUNK, _CHUNK)],
                ssem.at[b],
            )

        def wait_store(b):
            pltpu.make_async_copy(
                rows_v.at[b], out_hbm.at[pl.ds(out_row0, _CHUNK)], ssem.at[b]
            ).wait()

        start_gathers(0, 0)

        @pl.loop(0, n_chunks, step=_NBUF)
        def _chunks(lv):
            for p in range(_NBUF):
                c = lv + p
                c_fin = c - 2
                bf = (p - 2) % _NBUF

                @pl.when(c_fin >= 0)
                def _():
                    wait_gathers(bf)
                    start_store(c_fin, bf)

                c_new = c + 1
                bn = (p + 1) % _NBUF

                @pl.when(c_new < n_chunks)
                def _():
                    @pl.when(c_new >= _NBUF)
                    def _():
                        wait_store(bn)

                    start_gathers(c_new, bn)

        for c_fin in (n_chunks - 2, n_chunks - 1):
            bf = c_fin % _NBUF
            wait_gathers(bf)
            start_store(c_fin, bf)
        for b in range(_NBUF):
            wait_store(b)

    return gather_kernel


@functools.cache
def _build_transpose(batch, fields):
    def body(rows_ref, out_ref):
        t = jnp.transpose(rows_ref[...])
        out_ref[...] = t.reshape(1, _DIM // 8, 1, 8, _BBLK)

    return pl.pallas_call(
        body,
        grid=(fields, batch // _BBLK),
        in_specs=[
            pl.BlockSpec(
                (_BBLK, _DIM), lambda f, i: (f * (batch // _BBLK) + i, 0)
            ),
        ],
        out_specs=pl.BlockSpec(
            (1, _DIM // 8, 1, 8, _BBLK), lambda f, i: (f, 0, i, 0, 0)
        ),
        out_shape=jax.ShapeDtypeStruct(
            (fields, _DIM // 8, batch // _BBLK, 8, _BBLK), jnp.float32
        ),
    )


def kernel(x, weight):
    b, f = x.shape
    n_total = b * f
    xt = jnp.swapaxes(x, 0, 1).astype(jnp.int32)
    idx2d = xt.reshape(n_total // _IDX_LANES, _IDX_LANES)
    rows = _build_gather(n_total)(idx2d, weight)
    out5 = _build_transpose(b, f)(rows)
    return jnp.transpose(out5, (2, 4, 0, 1, 3)).reshape(b, f, _DIM)


# speedup vs baseline: 10.5426x; 2.7443x over previous
import functools

import jax
import jax.numpy as jnp
from jax import lax
from jax.experimental import pallas as pl
from jax.experimental.pallas import tpu as pltpu
from jax.experimental.pallas import tpu_sc as plsc

_DIM = 64
_IDX_LANES = 128
_CHUNK = 208
_NBUF = 4
_BBLK = 128


@functools.cache
def _build_gather(n_total):
    info = plsc.get_sparse_core_info()
    nc, ns = info.num_cores, info.num_subcores
    nw = nc * ns
    rows_per_w = n_total // nw
    n_chunks = rows_per_w // _CHUNK
    idx_rows_per_w = rows_per_w // _IDX_LANES

    mesh = plsc.VectorSubcoreMesh(core_axis_name="c", subcore_axis_name="s")

    @functools.partial(
        pl.kernel,
        mesh=mesh,
        compiler_params=pltpu.CompilerParams(use_tc_tiling_on_sc=True),
        out_type=jax.ShapeDtypeStruct((n_total, _DIM), jnp.float32),
        scratch_types=[
            pltpu.VMEM((idx_rows_per_w, _IDX_LANES), jnp.int32),
            pltpu.VMEM((_NBUF, _CHUNK, _DIM), jnp.float32),
            pltpu.SemaphoreType.DMA((_NBUF,)),
            pltpu.SemaphoreType.DMA((_NBUF,)),
        ],
    )
    def gather_kernel(idx_hbm, table_hbm, out_hbm, idx_all, rows_v, gsem, ssem):
        wid = lax.axis_index("s") * nc + lax.axis_index("c")
        idx_row0 = wid * idx_rows_per_w
        out_row0 = wid * rows_per_w
        pltpu.sync_copy(idx_hbm.at[pl.ds(idx_row0, idx_rows_per_w)], idx_all)

        def start_gathers(c, b):
            @pl.loop(0, _CHUNK // 16)
            def _grp(g):
                flat = c * _CHUNK + g * 16
                vec = idx_all[flat // _IDX_LANES,
                              pl.ds(flat % _IDX_LANES, 16)]
                for i in range(16):
                    pltpu.async_copy(
                        table_hbm.at[pl.ds(vec[i], 1)],
                        rows_v.at[b, pl.ds(g * 16 + i, 1)],
                        gsem.at[b],
                    )

        def wait_gathers(b):
            pltpu.make_async_copy(
                table_hbm.at[pl.ds(0, _CHUNK)], rows_v.at[b], gsem.at[b]
            ).wait()

        def start_store(c, b):
            return pltpu.async_copy(
                rows_v.at[b],
                out_hbm.at[pl.ds(out_row0 + c * _CHUNK, _CHUNK)],
                ssem.at[b],
            )

        def wait_store(b):
            pltpu.make_async_copy(
                rows_v.at[b], out_hbm.at[pl.ds(out_row0, _CHUNK)], ssem.at[b]
            ).wait()

        start_gathers(0, 0)

        @pl.loop(0, n_chunks, step=_NBUF)
        def _chunks(lv):
            for p in range(_NBUF):
                c = lv + p
                c_fin = c - 2
                bf = (p - 2) % _NBUF

                @pl.when(c_fin >= 0)
                def _():
                    wait_gathers(bf)
                    start_store(c_fin, bf)

                c_new = c + 1
                bn = (p + 1) % _NBUF

                @pl.when(c_new < n_chunks)
                def _():
                    @pl.when(c_new >= _NBUF)
                    def _():
                        wait_store(bn)

                    start_gathers(c_new, bn)

        for c_fin in (n_chunks - 2, n_chunks - 1):
            bf = c_fin % _NBUF
            wait_gathers(bf)
            start_store(c_fin, bf)
        for b in range(_NBUF):
            wait_store(b)

    return gather_kernel


@functools.cache
def _build_transpose(batch, fields):
    rows_per_blk = _BBLK * fields

    def body(rows_ref, out_ref):
        a = rows_ref[...].reshape(_BBLK, fields, _DIM)
        t = jnp.transpose(a, (1, 2, 0))
        out_ref[...] = t.reshape(fields, _DIM // 8, 1, 8, _BBLK)

    return pl.pallas_call(
        body,
        grid=(batch // _BBLK,),
        in_specs=[
            pl.BlockSpec((rows_per_blk, _DIM), lambda i: (i, 0)),
        ],
        out_specs=pl.BlockSpec(
            (fields, _DIM // 8, 1, 8, _BBLK), lambda i: (0, 0, i, 0, 0)
        ),
        out_shape=jax.ShapeDtypeStruct(
            (fields, _DIM // 8, batch // _BBLK, 8, _BBLK), jnp.float32
        ),
    )


def kernel(x, weight):
    b, f = x.shape
    n_total = b * f
    idx2d = x.reshape(n_total // _IDX_LANES, _IDX_LANES).astype(jnp.int32)
    rows = _build_gather(n_total)(idx2d, weight)
    out5 = _build_transpose(b, f)(rows)
    return jnp.transpose(out5, (2, 4, 0, 1, 3)).reshape(b, f, _DIM)
